# Initial kernel scaffold; baseline (speedup 1.0000x reference)
import jax
import jax.numpy as jnp
from jax.experimental import pallas as pl


def kernel(x, edge_index, W1, att_src1, att_dst1, b1, W2, att_src2, att_dst2, b2, Wo, bo):
    raise NotImplementedError("write your pallas kernel here")


# speedup vs baseline: 8.5533x; 8.5533x over previous
import functools

import jax
import jax.numpy as jnp
from jax import lax
from jax.experimental import pallas as pl
from jax.experimental.pallas import tpu as pltpu
from jax.experimental.pallas import tpu_sc as plsc

_N = 10000
_E = 320000
_H = 8
_C = 128
_HC = _H * _C

_NC = 2
_NS = 16

_K1 = 16
_EP1 = _E // _NS
_IT1 = _EP1 // _K1
_K2 = 8
_EP2 = _E // (_NC * _NS)
_IT2 = _EP2 // _K2
_RCH = 8
_NRC = _N // _RCH
_RIT = -(-_NRC // _NS)

_RB = 400


def _edge_body(t_hbm, h_hbm, src_hbm, dst_hbm, out_hbm,
               t_sh, denom_sh, acc_sh,
               src1_v, dst1_v, ts_v, td_v, ex_v,
               src2_v, dst2_v, hrows_v, ts2_v, td2_v, den_v, attn_v, msg_v,
               zd_v, stage_v):
    c = lax.axis_index("c")
    s = lax.axis_index("s")
    zero16 = jnp.zeros((16,), jnp.float32)
    rot8 = (lax.iota(jnp.int32, 16) + 8) % 16

    def zrow_d(i, carry):
        zd_v[i, :] = zero16
        return carry

    lax.fori_loop(0, _RCH, zrow_d, None)

    def zrow_a(i, carry):
        for j in range(_C // 16):
            stage_v[i, pl.ds(j * 16, 16)] = zero16
        return carry

    lax.fori_loop(0, _RCH, zrow_a, None)

    def zcopy(i, carry):
        k = s + i * _NS

        @pl.when(k < _NRC)
        def _():
            r = pl.ds(k * _RCH, _RCH)
            pltpu.sync_copy(zd_v, denom_sh.at[r])
            pltpu.sync_copy(stage_v, acc_sh.at[r])
            pltpu.sync_copy(t_hbm.at[r], t_sh.at[r])

        return carry

    lax.fori_loop(0, _RIT, zcopy, None)
    plsc.subcore_barrier()

    def p1(g, carry):
        base = s * _EP1 + g * _K1
        pltpu.sync_copy(src_hbm.at[pl.ds(base, _K1)], src1_v)
        pltpu.sync_copy(dst_hbm.at[pl.ds(base, _K1)], dst1_v)
        pltpu.sync_copy(t_sh.at[src1_v], ts_v)
        pltpu.sync_copy(t_sh.at[dst1_v], td_v)

        def ew(i, cc):
            v = ts_v[i, :] + td_v[i, :][rot8]
            ex_v[i, :] = jnp.exp(jnp.maximum(v, 0.2 * v))
            return cc

        lax.fori_loop(0, _K1, ew, None)
        pltpu.sync_copy(ex_v, denom_sh.at[dst1_v], add=True)
        return carry

    lax.fori_loop(0, _IT1, p1, None)
    plsc.subcore_barrier()

    def p2(g, carry):
        base = c * (_E // _NC) + s * _EP2 + g * _K2
        pltpu.sync_copy(src_hbm.at[pl.ds(base, _K2)], src2_v)
        pltpu.sync_copy(dst_hbm.at[pl.ds(base, _K2)], dst2_v)
        pltpu.sync_copy(h_hbm.at[src2_v], hrows_v)
        pltpu.sync_copy(t_sh.at[src2_v], ts2_v)
        pltpu.sync_copy(t_sh.at[dst2_v], td2_v)
        pltpu.sync_copy(denom_sh.at[dst2_v], den_v)

        def ew2(i, cc):
            v = ts2_v[i, :] + td2_v[i, :][rot8]
            ex = jnp.exp(jnp.maximum(v, 0.2 * v))
            attn_v[pl.ds(i * 16, 16)] = ex / (den_v[i, :] + 1e-16)
            return cc

        lax.fori_loop(0, _K2, ew2, None)

        def per_edge(k, cc):
            arow = attn_v[pl.ds(k * 16, 16)]
            accs = [zero16] * (_C // 16)
            for h in range(_H):
                a = jnp.full((16,), arow[h], jnp.float32)
                accs = [accs[j] + a * hrows_v[k, pl.ds(h * _C + j * 16, 16)]
                        for j in range(_C // 16)]
            for j in range(_C // 16):
                msg_v[k, pl.ds(j * 16, 16)] = accs[j]
            return cc

        lax.fori_loop(0, _K2, per_edge, None)
        pltpu.sync_copy(msg_v, acc_sh.at[dst2_v], add=True)
        return carry

    lax.fori_loop(0, _IT2, p2, None)
    plsc.subcore_barrier()

    def wout(i, carry):
        k = s + i * _NS

        @pl.when(k < _NRC)
        def _():
            r = pl.ds(k * _RCH, _RCH)
            pltpu.sync_copy(acc_sh.at[r], stage_v)
            pltpu.sync_copy(stage_v, out_hbm.at[c, r])

        return carry

    lax.fori_loop(0, _RIT, wout, None)


@functools.cache
def _edge_call():
  return pl.kernel(
    _edge_body,
    out_type=jax.ShapeDtypeStruct((_NC, _N, _C), jnp.float32),
    mesh=plsc.VectorSubcoreMesh(core_axis_name="c", subcore_axis_name="s",
                                num_cores=_NC, num_subcores=_NS),
    scratch_types=[
        pltpu.VMEM_SHARED((_N, 16), jnp.float32),
        pltpu.VMEM_SHARED((_N, 16), jnp.float32),
        pltpu.VMEM_SHARED((_N, _C), jnp.float32),
        pltpu.VMEM((_K1,), jnp.int32),
        pltpu.VMEM((_K1,), jnp.int32),
        pltpu.VMEM((_K1, 16), jnp.float32),
        pltpu.VMEM((_K1, 16), jnp.float32),
        pltpu.VMEM((_K1, 16), jnp.float32),
        pltpu.VMEM((_K2,), jnp.int32),
        pltpu.VMEM((_K2,), jnp.int32),
        pltpu.VMEM((_K2, _HC), jnp.float32),
        pltpu.VMEM((_K2, 16), jnp.float32),
        pltpu.VMEM((_K2, 16), jnp.float32),
        pltpu.VMEM((_K2, 16), jnp.float32),
        pltpu.VMEM((_K2 * 16,), jnp.float32),
        pltpu.VMEM((_K2, _C), jnp.float32),
        pltpu.VMEM((_RCH, 16), jnp.float32),
        pltpu.VMEM((_RCH, _C), jnp.float32),
    ],
  )


def _mm_first(x, W, Wt):
    def body(x_ref, w_ref, wt_ref, h_ref, t_ref):
        xb = x_ref[...]
        h_ref[...] = jnp.dot(xb, w_ref[...], preferred_element_type=jnp.float32)
        t_ref[...] = jnp.dot(xb, wt_ref[...], preferred_element_type=jnp.float32)

    return pl.pallas_call(
        body,
        grid=(_N // _RB,),
        in_specs=[
            pl.BlockSpec((_RB, _C), lambda i: (i, 0)),
            pl.BlockSpec((_C, _HC), lambda i: (0, 0)),
            pl.BlockSpec((_C, 16), lambda i: (0, 0)),
        ],
        out_specs=[
            pl.BlockSpec((_RB, _HC), lambda i: (i, 0)),
            pl.BlockSpec((_RB, 16), lambda i: (i, 0)),
        ],
        out_shape=[
            jax.ShapeDtypeStruct((_N, _HC), jnp.float32),
            jax.ShapeDtypeStruct((_N, 16), jnp.float32),
        ],
    )(x, W, Wt)


def _mm_mid(p0, p1, b, W, Wt):
    def body(p0_ref, p1_ref, b_ref, w_ref, wt_ref, h_ref, t_ref):
        xb = (p0_ref[...] + p1_ref[...]) * (1.0 / _H) + b_ref[...]
        xb = jnp.where(xb > 0, xb, jnp.exp(xb) - 1.0)
        h_ref[...] = jnp.dot(xb, w_ref[...], preferred_element_type=jnp.float32)
        t_ref[...] = jnp.dot(xb, wt_ref[...], preferred_element_type=jnp.float32)

    return pl.pallas_call(
        body,
        grid=(_N // _RB,),
        in_specs=[
            pl.BlockSpec((_RB, _C), lambda i: (i, 0)),
            pl.BlockSpec((_RB, _C), lambda i: (i, 0)),
            pl.BlockSpec((1, _C), lambda i: (0, 0)),
            pl.BlockSpec((_C, _HC), lambda i: (0, 0)),
            pl.BlockSpec((_C, 16), lambda i: (0, 0)),
        ],
        out_specs=[
            pl.BlockSpec((_RB, _HC), lambda i: (i, 0)),
            pl.BlockSpec((_RB, 16), lambda i: (i, 0)),
        ],
        out_shape=[
            jax.ShapeDtypeStruct((_N, _HC), jnp.float32),
            jax.ShapeDtypeStruct((_N, 16), jnp.float32),
        ],
    )(p0, p1, b, W, Wt)


def _mm_final(q0, q1, b, Wo, bo):
    def body(q0_ref, q1_ref, b_ref, wo_ref, bo_ref, o_ref):
        xb = (q0_ref[...] + q1_ref[...]) * (1.0 / _H) + b_ref[...]
        xb = jnp.where(xb > 0, xb, jnp.exp(xb) - 1.0)
        o_ref[...] = (jnp.dot(xb, wo_ref[...],
                              preferred_element_type=jnp.float32)
                      + bo_ref[...])

    return pl.pallas_call(
        body,
        grid=(_N // _RB,),
        in_specs=[
            pl.BlockSpec((_RB, _C), lambda i: (i, 0)),
            pl.BlockSpec((_RB, _C), lambda i: (i, 0)),
            pl.BlockSpec((1, _C), lambda i: (0, 0)),
            pl.BlockSpec((_C, _C), lambda i: (0, 0)),
            pl.BlockSpec((1, _C), lambda i: (0, 0)),
        ],
        out_specs=pl.BlockSpec((_RB, _C), lambda i: (i, 0)),
        out_shape=jax.ShapeDtypeStruct((_N, _C), jnp.float32),
    )(q0, q1, b, Wo, bo)


def _fold(W, att):
    return jnp.einsum('dhc,hc->dh', W.reshape(W.shape[0], _H, _C), att[0])


def kernel(x, edge_index, W1, att_src1, att_dst1, b1,
           W2, att_src2, att_dst2, b2, Wo, bo):
    src = edge_index[0].astype(jnp.int32)
    dst = edge_index[1].astype(jnp.int32)
    Wt1 = jnp.concatenate([_fold(W1, att_src1), _fold(W1, att_dst1)], axis=1)
    Wt2 = jnp.concatenate([_fold(W2, att_src2), _fold(W2, att_dst2)], axis=1)

    h1, T1 = _mm_first(x, W1, Wt1)
    agg1 = _edge_call()(T1, h1, src, dst)
    h2, T2 = _mm_mid(agg1[0], agg1[1], b1.reshape(1, -1), W2, Wt2)
    agg2 = _edge_call()(T2, h2, src, dst)
    return _mm_final(agg2[0], agg2[1], b2.reshape(1, -1), Wo,
                     bo.reshape(1, -1))
